# baseline (speedup 1.0000x reference)
import functools
import math

import jax
import jax.numpy as jnp
from jax import lax
from jax.experimental import pallas as pl
from jax.experimental.pallas import tpu as pltpu
from jax.experimental.pallas import tpu_sc as plsc

N_NODES = 10000
N_EDGES = 160000
N_TRIP = 160000
N_GRAPHS = 512
H = 128
R = 6
S = 7
INT_EMB = 64
BD = 8
BA = 8
BT = 8
OUT_EMB = 128
OUT_DIM = 1
CUTOFF = 10.0
P_ENV = 5
NUM_LAYERS = 4


def _swish(x):
    return x * jax.nn.sigmoid(x)


_CS = 26752
_BUF = 26880
_TBE = 80


def _seg_edges_body(m_hbm, ji_hbm, out_hbm, mv, jv, jadj, zbuf, shared):
    cid = lax.axis_index("c")
    sid = lax.axis_index("s")

    def zrow(r, carry):
        for k in range(4):
            zbuf[r, pl.ds(k * 16, 16)] = jnp.zeros((16,), jnp.float32)
        return carry
    lax.fori_loop(0, _TBE, zrow, 0)

    def do_chunk(c, carry):
        cbase = (cid * 3 + c) * _CS
        z0 = sid * 1680
        for t in range(21):
            pltpu.sync_copy(zbuf, shared.at[pl.ds(z0 + t * 80, 80)])
        plsc.subcore_barrier()
        t0 = sid * 10000

        def blk(b, carry2):
            base = t0 + b * _TBE
            pltpu.sync_copy(ji_hbm.at[pl.ds(base, _TBE)], jv)
            pltpu.sync_copy(m_hbm.at[pl.ds(base, _TBE)], mv)
            for k in range(_TBE // 16):
                v = jv[pl.ds(k * 16, 16)] - cbase
                inb = (v >= 0) & (v < _CS)
                jadj[pl.ds(k * 16, 16)] = jnp.where(inb, v, _CS)
            pltpu.sync_copy(mv, shared.at[jadj], add=True)
            return carry2
        lax.fori_loop(0, 10000 // _TBE, blk, 0)
        plsc.subcore_barrier()
        d0 = sid * 1672
        pltpu.sync_copy(shared.at[pl.ds(d0, 1672)],
                        out_hbm.at[pl.ds(cbase + d0, 1672)])
        plsc.subcore_barrier()
        return carry
    lax.fori_loop(0, 3, do_chunk, 0)


_seg_edges_call = pl.kernel(
    _seg_edges_body,
    out_type=jax.ShapeDtypeStruct((6 * _CS, 64), jnp.float32),
    mesh=plsc.VectorSubcoreMesh(core_axis_name="c", subcore_axis_name="s"),
    scratch_types=[
        pltpu.VMEM((_TBE, 64), jnp.float32),
        pltpu.VMEM((_TBE,), jnp.int32),
        pltpu.VMEM((_TBE,), jnp.int32),
        pltpu.VMEM((80, 64), jnp.float32),
        pltpu.VMEM_SHARED((_BUF, 64), jnp.float32),
    ],
    compiler_params=pltpu.CompilerParams(use_tc_tiling_on_sc=False),
)


def _seg_edges(m, ji):
    return _seg_edges_call(m, ji)[:N_EDGES]


def _envelope(x):
    p = P_ENV + 1
    a = -(p + 1) * (p + 2) / 2.0
    b = p * (p + 2)
    c = -p * (p + 1) / 2.0
    x4 = (x * x) * (x * x)
    return 1.0 / x + a * x4 * x + b * x4 * x * x + c * x4 * x * x * x


def _dist_emb(dist):
    x = jnp.clip(dist / CUTOFF, 1e-4, None)
    freqs = jnp.arange(1, R + 1, dtype=jnp.float32) * math.pi
    return _envelope(x)[:, None] * jnp.sin(freqs[None, :] * x[:, None])


def _sph_jl_all(x):
    out = []
    for l in range(S):
        z = jnp.clip(x[l], 0.1, None)
        sz = jnp.sin(z)
        cz = jnp.cos(z)
        j0 = sz / z
        if l == 0:
            out.append(j0)
            continue
        j1 = sz / (z * z) - cz / z
        jm, jc = j0, j1
        for ll in range(2, l + 1):
            jm, jc = jc, (2.0 * ll - 1.0) / z * jc - jm
        out.append(jc)
    return out


def _base42(dist_t, ct):
    x = jnp.clip(dist_t / CUTOFF, 1e-4, None)
    ps = [jnp.ones_like(ct), ct]
    for l in range(2, S):
        ps.append(((2.0 * l - 1.0) * ct * ps[l - 1] - (l - 1.0) * ps[l - 2]) / l)
    zs = []
    for l in range(S):
        roots = (jnp.arange(1, R + 1, dtype=jnp.float32) + 0.5 * l) * math.pi
        zs.append(roots[None, :] * x[:, None])
    jls = _sph_jl_all(zs)
    feats = [jls[l] * ps[l][:, None] for l in range(S)]
    return jnp.concatenate(feats, axis=1)


def _update_e(p, e1, rbf, base42, cosm, idx_kj, idx_ji, want_e2):
    x_ji = _swish(e1 @ p['lin_ji']['w'] + p['lin_ji']['b'])
    x_kj = _swish(e1 @ p['lin_kj']['w'] + p['lin_kj']['b'])
    rbf_m = (rbf @ p['lin_rbf1']['w']) @ p['lin_rbf2']['w']
    x_kj = x_kj * rbf_m
    h = _swish(x_kj @ p['lin_down']['w'])
    sb = (base42 @ p['lin_sbf1']['w']) @ p['lin_sbf2']['w']
    w1 = p['lin_t1']['w'].reshape(S, S, R, BT).transpose(0, 2, 1, 3).reshape(S * R, S * BT)
    tp = (base42 @ w1).reshape(-1, S, BT)
    tb8 = jnp.einsum('nm,nmk->nk', cosm, tp)
    tb = tb8 @ p['lin_t2']['w']
    m = h[idx_kj] * sb * tb
    agg = _seg_edges(m, idx_ji)
    x2 = _swish(agg @ p['lin_up']['w'])
    e1n = x_ji + x2
    for l1, l2 in p['before_skip']:
        e1n = e1n + _swish(_swish(e1n @ l1['w'] + l1['b']) @ l2['w'] + l2['b'])
    e1n = _swish(e1n @ p['lin_mid']['w'] + p['lin_mid']['b']) + e1
    for l1, l2 in p['after_skip']:
        e1n = e1n + _swish(_swish(e1n @ l1['w'] + l1['b']) @ l2['w'] + l2['b'])
    e2 = (rbf @ p['lin_rbf']['w']) * e1n if want_e2 else None
    return e1n, e2


def kernel(atoms, pos, batch, edge_index, idx_kj, idx_ji, idx_t, params):
    j_idx = edge_index[0]
    i_idx = edge_index[1]
    vecs = pos[j_idx] - pos[i_idx]
    dist = jnp.sqrt(jnp.sum(vecs ** 2, axis=-1) + 1e-12)
    pos_ji = vecs[idx_ji]
    pos_kj = vecs[idx_kj]
    ref_v = vecs[idx_t]
    a = jnp.sum(pos_ji * pos_kj, axis=-1)
    n1 = jnp.cross(pos_ji, pos_kj)
    b = jnp.sqrt(jnp.sum(n1 ** 2, axis=-1) + 1e-12)
    ct = a / jnp.sqrt(a * a + b * b)
    n2 = jnp.cross(pos_ji, ref_v)
    dist_ji = jnp.sqrt(jnp.sum(pos_ji ** 2, axis=-1) + 1e-12)
    t_b = jnp.sum(jnp.cross(n1, n2) * pos_ji, axis=-1) / dist_ji + 1e-6
    t_a = jnp.sum(n1 * n2, axis=-1) + 1e-6
    cphi = t_a / jnp.sqrt(t_a * t_a + t_b * t_b + 1e-30)
    cs = [jnp.ones_like(cphi), cphi]
    for m in range(2, S):
        cs.append(2.0 * cphi * cs[m - 1] - cs[m - 2])
    cosm = jnp.stack(cs, axis=1)

    rbf = _dist_emb(dist)
    dist_t = jnp.sqrt(jnp.sum(pos_kj ** 2, axis=-1) + 1e-12)
    base42 = _base42(dist_t, ct)

    x = params['node_emb'][atoms]
    pi_ = params['init']
    rbf0 = _swish(rbf @ pi_['rbf0']['w'] + pi_['rbf0']['b'])
    wcat = pi_['lin']['w']
    e1 = _swish(x[i_idx] @ wcat[:H] + x[j_idx] @ wcat[H:2 * H]
                + rbf0 @ wcat[2 * H:] + pi_['lin']['b'])

    e2 = None
    for layer in range(NUM_LAYERS):
        e1, e2 = _update_e(params['update_es'][layer], e1, rbf, base42, cosm,
                           idx_kj, idx_ji, want_e2=(layer == NUM_LAYERS - 1))

    pv = params['update_vs'][NUM_LAYERS - 1]
    v = jax.ops.segment_sum(e2, i_idx, num_segments=N_NODES)
    v = _swish(v @ pv['lin_up']['w'] + pv['lin_up']['b'])
    for lp in pv['lins']:
        v = _swish(v @ lp['w'] + lp['b'])
    v = v @ pv['lin']['w']
    return jax.ops.segment_sum(v, batch, num_segments=N_GRAPHS)


# speedup vs baseline: 1.1947x; 1.1947x over previous
import functools
import math

import jax
import jax.numpy as jnp
from jax import lax
from jax.experimental import pallas as pl
from jax.experimental.pallas import tpu as pltpu
from jax.experimental.pallas import tpu_sc as plsc

N_NODES = 10000
N_EDGES = 160000
N_TRIP = 160000
N_GRAPHS = 512
H = 128
R = 6
S = 7
INT_EMB = 64
BD = 8
BA = 8
BT = 8
OUT_EMB = 128
OUT_DIM = 1
CUTOFF = 10.0
P_ENV = 5
NUM_LAYERS = 4


def _swish(x):
    return x * jax.nn.sigmoid(x)


_CS = 26752
_BUF = 26880
_TBE = 80
_CAP = 5000


def _part_body(ji_hbm, kj_hbm, kreg_hbm, treg_hbm, dreg_hbm, cnt_hbm,
               jb, kb, regk, regt, regd, cbuf):
    cid = lax.axis_index("c")
    sid = lax.axis_index("s")
    w = cid * 16 + sid

    def initrow(i, carry):
        regk[pl.ds(i * 16, 16)] = jnp.zeros((16,), jnp.int32)
        regt[pl.ds(i * 16, 16)] = jnp.zeros((16,), jnp.int32)
        regd[pl.ds(i * 16, 16)] = jnp.full((16,), _CS, jnp.int32)
        return carry
    lax.fori_loop(0, 6 * _CAP // 16, initrow, 0)

    lanes = lax.iota(jnp.int32, 16)

    def blk(b, cnts):
        base = w * _CAP + b * _TBE
        pltpu.sync_copy(ji_hbm.at[pl.ds(base, _TBE)], jb)
        pltpu.sync_copy(kj_hbm.at[pl.ds(base, _TBE)], kb)
        new = list(cnts)
        for k in range(_TBE // 16):
            jv = jb[pl.ds(k * 16, 16)]
            kv = kb[pl.ds(k * 16, 16)]
            tv = lanes + (base + k * 16)
            for c in range(6):
                loc = jv - c * _CS
                msk = (loc >= 0) & (loc < _CS)
                mi = msk.astype(jnp.int32)
                slot = c * _CAP + new[c] + plsc.cumsum(mi) - 1
                plsc.store_scatter(regk, [slot], kv, mask=msk)
                plsc.store_scatter(regt, [slot], tv, mask=msk)
                plsc.store_scatter(regd, [slot], loc, mask=msk)
                new[c] = new[c] + jnp.sum(mi)
        return tuple(new)
    z = jnp.int32(0)
    cnts = lax.fori_loop(0, _CAP // _TBE, blk, (z, z, z, z, z, z))

    cv = jnp.zeros((16,), jnp.int32)
    for c in range(6):
        cv = jnp.where(lanes == c, cnts[c], cv)
    cbuf[pl.ds(0, 16)] = cv
    pltpu.sync_copy(cbuf, cnt_hbm.at[w])
    pltpu.sync_copy(regk, kreg_hbm.at[w])
    pltpu.sync_copy(regt, treg_hbm.at[w])
    pltpu.sync_copy(regd, dreg_hbm.at[w])


_part_call = pl.kernel(
    _part_body,
    out_type=(
        jax.ShapeDtypeStruct((32, 6 * _CAP), jnp.int32),
        jax.ShapeDtypeStruct((32, 6 * _CAP), jnp.int32),
        jax.ShapeDtypeStruct((32, 6 * _CAP), jnp.int32),
        jax.ShapeDtypeStruct((32, 16), jnp.int32),
    ),
    mesh=plsc.VectorSubcoreMesh(core_axis_name="c", subcore_axis_name="s"),
    scratch_types=[
        pltpu.VMEM((_TBE,), jnp.int32),
        pltpu.VMEM((_TBE,), jnp.int32),
        pltpu.VMEM((6 * _CAP,), jnp.int32),
        pltpu.VMEM((6 * _CAP,), jnp.int32),
        pltpu.VMEM((6 * _CAP,), jnp.int32),
        pltpu.VMEM((16,), jnp.int32),
    ],
    compiler_params=pltpu.CompilerParams(use_tc_tiling_on_sc=False, needs_layout_passes=False),
)


def _agg_body(h_hbm, s_hbm, kreg_hbm, treg_hbm, dreg_hbm, cnt_hbm, out_hbm,
              kb, tb_, db, hv, sv, zbuf, cntv, sem, shared):
    cid = lax.axis_index("c")
    sid = lax.axis_index("s")

    def zrow(r, carry):
        for q in range(4):
            zbuf[r, pl.ds(q * 16, 16)] = jnp.zeros((16,), jnp.float32)
        return carry
    lax.fori_loop(0, _TBE, zrow, 0)

    def do_chunk(c, carry):
        chunk = cid * 3 + c
        cbase = chunk * _CS
        z0 = sid * 1680
        for t in range(21):
            pltpu.sync_copy(zbuf, shared.at[pl.ds(z0 + t * 80, 80)])
        plsc.subcore_barrier()

        def do_region(rr, carry2):
            w2 = sid * 2 + rr
            pltpu.sync_copy(cnt_hbm.at[w2], cntv)
            lanes = lax.iota(jnp.int32, 16)
            cnt = jnp.sum(jnp.where(lanes == chunk, cntv[pl.ds(0, 16)], 0))
            nb = (cnt + (_TBE - 1)) // _TBE

            def blk(b, carry3):
                o = chunk * _CAP + b * _TBE
                pltpu.sync_copy(kreg_hbm.at[w2, pl.ds(o, _TBE)], kb)
                pltpu.sync_copy(treg_hbm.at[w2, pl.ds(o, _TBE)], tb_)
                pltpu.sync_copy(dreg_hbm.at[w2, pl.ds(o, _TBE)], db)
                pltpu.async_copy(h_hbm.at[kb], hv, sem).wait()
                pltpu.async_copy(s_hbm.at[tb_], sv, sem).wait()

                def mrow(r, carry4):
                    for q in range(4):
                        hv[r, pl.ds(q * 16, 16)] = (
                            hv[r, pl.ds(q * 16, 16)] * sv[r, pl.ds(q * 16, 16)])
                    return carry4
                lax.fori_loop(0, _TBE, mrow, 0)
                pltpu.sync_copy(hv, shared.at[db], add=True)
                return carry3
            lax.fori_loop(0, nb, blk, 0)
            return carry2
        lax.fori_loop(0, 2, do_region, 0)
        plsc.subcore_barrier()
        d0 = sid * 1672
        pltpu.sync_copy(shared.at[pl.ds(d0, 1672)],
                        out_hbm.at[pl.ds(cbase + d0, 1672)])
        plsc.subcore_barrier()
        return carry
    lax.fori_loop(0, 3, do_chunk, 0)


_agg_call = pl.kernel(
    _agg_body,
    out_type=jax.ShapeDtypeStruct((6 * _CS, 64), jnp.float32),
    mesh=plsc.VectorSubcoreMesh(core_axis_name="c", subcore_axis_name="s"),
    scratch_types=[
        pltpu.VMEM((_TBE,), jnp.int32),
        pltpu.VMEM((_TBE,), jnp.int32),
        pltpu.VMEM((_TBE,), jnp.int32),
        pltpu.VMEM((_TBE, 64), jnp.float32),
        pltpu.VMEM((_TBE, 64), jnp.float32),
        pltpu.VMEM((80, 64), jnp.float32),
        pltpu.VMEM((16,), jnp.int32),
        pltpu.SemaphoreType.DMA,
        pltpu.VMEM_SHARED((_BUF, 64), jnp.float32),
    ],
    compiler_params=pltpu.CompilerParams(use_tc_tiling_on_sc=False, needs_layout_passes=False),
)


_NROWS = 10240


def _seg_nodes_body(e_hbm, i_hbm, out_hbm, iv, ev, zbuf, shared):
    cid = lax.axis_index("c")
    sid = lax.axis_index("s")

    def zrow(r, carry):
        for q in range(8):
            zbuf[r, pl.ds(q * 16, 16)] = jnp.zeros((16,), jnp.float32)
        return carry
    lax.fori_loop(0, 40, zrow, 0)
    z0 = sid * 640
    for t in range(16):
        pltpu.sync_copy(zbuf, shared.at[pl.ds(z0 + t * 40, 40)])
    plsc.subcore_barrier()
    w = cid * 16 + sid
    t0 = w * 5000

    def blk(b, carry):
        base = t0 + b * 40
        pltpu.sync_copy(i_hbm.at[pl.ds(base, 40)], iv)
        pltpu.sync_copy(e_hbm.at[pl.ds(base, 40)], ev)
        pltpu.sync_copy(ev, shared.at[iv], add=True)
        return carry
    lax.fori_loop(0, 125, blk, 0)
    plsc.subcore_barrier()
    pltpu.sync_copy(shared.at[pl.ds(z0, 640)], out_hbm.at[cid, pl.ds(z0, 640)])


_seg_nodes_call = pl.kernel(
    _seg_nodes_body,
    out_type=jax.ShapeDtypeStruct((2, _NROWS, 128), jnp.float32),
    mesh=plsc.VectorSubcoreMesh(core_axis_name="c", subcore_axis_name="s"),
    scratch_types=[
        pltpu.VMEM((40,), jnp.int32),
        pltpu.VMEM((40, 128), jnp.float32),
        pltpu.VMEM((40, 128), jnp.float32),
        pltpu.VMEM_SHARED((_NROWS, 128), jnp.float32),
    ],
    compiler_params=pltpu.CompilerParams(use_tc_tiling_on_sc=False, needs_layout_passes=False),
)


def _seg_nodes(e2, i_idx):
    out = _seg_nodes_call(e2, i_idx)
    return (out[0] + out[1])[:N_NODES]


def _envelope(x):
    p = P_ENV + 1
    a = -(p + 1) * (p + 2) / 2.0
    b = p * (p + 2)
    c = -p * (p + 1) / 2.0
    x4 = (x * x) * (x * x)
    return 1.0 / x + a * x4 * x + b * x4 * x * x + c * x4 * x * x * x


def _dist_emb(dist):
    x = jnp.clip(dist / CUTOFF, 1e-4, None)
    freqs = jnp.arange(1, R + 1, dtype=jnp.float32) * math.pi
    return _envelope(x)[:, None] * jnp.sin(freqs[None, :] * x[:, None])


def _sph_jl_all(x):
    out = []
    for l in range(S):
        z = jnp.clip(x[l], 0.1, None)
        sz = jnp.sin(z)
        cz = jnp.cos(z)
        j0 = sz / z
        if l == 0:
            out.append(j0)
            continue
        j1 = sz / (z * z) - cz / z
        jm, jc = j0, j1
        for ll in range(2, l + 1):
            jm, jc = jc, (2.0 * ll - 1.0) / z * jc - jm
        out.append(jc)
    return out


def _base42(dist_t, ct):
    x = jnp.clip(dist_t / CUTOFF, 1e-4, None)
    ps = [jnp.ones_like(ct), ct]
    for l in range(2, S):
        ps.append(((2.0 * l - 1.0) * ct * ps[l - 1] - (l - 1.0) * ps[l - 2]) / l)
    zs = []
    for l in range(S):
        roots = (jnp.arange(1, R + 1, dtype=jnp.float32) + 0.5 * l) * math.pi
        zs.append(roots[None, :] * x[:, None])
    jls = _sph_jl_all(zs)
    feats = [jls[l] * ps[l][:, None] for l in range(S)]
    return jnp.concatenate(feats, axis=1)


def _update_e(p, e1, rbf, base42, cosm, part, want_e2):
    x_ji = _swish(e1 @ p['lin_ji']['w'] + p['lin_ji']['b'])
    x_kj = _swish(e1 @ p['lin_kj']['w'] + p['lin_kj']['b'])
    rbf_m = (rbf @ p['lin_rbf1']['w']) @ p['lin_rbf2']['w']
    x_kj = x_kj * rbf_m
    h = _swish(x_kj @ p['lin_down']['w'])
    sb = (base42 @ p['lin_sbf1']['w']) @ p['lin_sbf2']['w']
    w1 = p['lin_t1']['w'].reshape(S, S, R, BT).transpose(0, 2, 1, 3).reshape(S * R, S * BT)
    tp = (base42 @ w1).reshape(-1, S, BT)
    tb8 = jnp.einsum('nm,nmk->nk', cosm, tp)
    tb = tb8 @ p['lin_t2']['w']
    s = sb * tb
    agg = _agg_call(h, s, *part)[:N_EDGES]
    x2 = _swish(agg @ p['lin_up']['w'])
    e1n = x_ji + x2
    for l1, l2 in p['before_skip']:
        e1n = e1n + _swish(_swish(e1n @ l1['w'] + l1['b']) @ l2['w'] + l2['b'])
    e1n = _swish(e1n @ p['lin_mid']['w'] + p['lin_mid']['b']) + e1
    for l1, l2 in p['after_skip']:
        e1n = e1n + _swish(_swish(e1n @ l1['w'] + l1['b']) @ l2['w'] + l2['b'])
    e2 = (rbf @ p['lin_rbf']['w']) * e1n if want_e2 else None
    return e1n, e2


def kernel(atoms, pos, batch, edge_index, idx_kj, idx_ji, idx_t, params):
    j_idx = edge_index[0]
    i_idx = edge_index[1]
    vecs = pos[j_idx] - pos[i_idx]
    dist = jnp.sqrt(jnp.sum(vecs ** 2, axis=-1) + 1e-12)
    pos_ji = vecs[idx_ji]
    pos_kj = vecs[idx_kj]
    ref_v = vecs[idx_t]
    a = jnp.sum(pos_ji * pos_kj, axis=-1)
    n1 = jnp.cross(pos_ji, pos_kj)
    b = jnp.sqrt(jnp.sum(n1 ** 2, axis=-1) + 1e-12)
    ct = a / jnp.sqrt(a * a + b * b)
    n2 = jnp.cross(pos_ji, ref_v)
    dist_ji = jnp.sqrt(jnp.sum(pos_ji ** 2, axis=-1) + 1e-12)
    t_b = jnp.sum(jnp.cross(n1, n2) * pos_ji, axis=-1) / dist_ji + 1e-6
    t_a = jnp.sum(n1 * n2, axis=-1) + 1e-6
    cphi = t_a / jnp.sqrt(t_a * t_a + t_b * t_b + 1e-30)
    cs = [jnp.ones_like(cphi), cphi]
    for m in range(2, S):
        cs.append(2.0 * cphi * cs[m - 1] - cs[m - 2])
    cosm = jnp.stack(cs, axis=1)

    rbf = _dist_emb(dist)
    dist_t = jnp.sqrt(jnp.sum(pos_kj ** 2, axis=-1) + 1e-12)
    base42 = _base42(dist_t, ct)

    x = params['node_emb'][atoms]
    pi_ = params['init']
    rbf0 = _swish(rbf @ pi_['rbf0']['w'] + pi_['rbf0']['b'])
    wcat = pi_['lin']['w']
    e1 = _swish(x[i_idx] @ wcat[:H] + x[j_idx] @ wcat[H:2 * H]
                + rbf0 @ wcat[2 * H:] + pi_['lin']['b'])

    part = _part_call(idx_ji.astype(jnp.int32), idx_kj.astype(jnp.int32))
    e2 = None
    for layer in range(NUM_LAYERS):
        e1, e2 = _update_e(params['update_es'][layer], e1, rbf, base42, cosm,
                           part, want_e2=(layer == NUM_LAYERS - 1))

    pv = params['update_vs'][NUM_LAYERS - 1]
    v = _seg_nodes(e2, i_idx.astype(jnp.int32))
    v = _swish(v @ pv['lin_up']['w'] + pv['lin_up']['b'])
    for lp in pv['lins']:
        v = _swish(v @ lp['w'] + lp['b'])
    v = v @ pv['lin']['w']
    return jax.ops.segment_sum(v, batch, num_segments=N_GRAPHS)
